# Initial kernel scaffold; baseline (speedup 1.0000x reference)
import jax
import jax.numpy as jnp
from jax.experimental import pallas as pl


def kernel(hidden_states, gate_w, w1, w3, w2):
    raise NotImplementedError("write your pallas kernel here")


# speedup vs baseline: 1.1673x; 1.1673x over previous
import functools

import jax
import jax.numpy as jnp
from jax.experimental import pallas as pl
from jax.experimental.pallas import tpu as pltpu

T = 2048
H = 1024
FFN = 2048
E = 8
EPAD = 128
BM = 1024


def _router_body(hs_ref, gw_ref, logits_ref, wtok_ref):
    x = hs_ref[...]
    l = jax.lax.dot_general(
        x, gw_ref[...], (((1,), (1,)), ((), ())),
        preferred_element_type=jnp.float32)
    logits_ref[...] = l
    lane = jax.lax.broadcasted_iota(jnp.int32, l.shape, 1)
    neg = jnp.float32(-1e30)
    big = jnp.int32(10**9)
    lm = jnp.where(lane < E, l, neg)
    m1 = jnp.max(lm, axis=1, keepdims=True)
    id1 = jnp.min(jnp.where(lm == m1, lane, big), axis=1, keepdims=True)
    lm2 = jnp.where(lane == id1, neg, lm)
    m2 = jnp.max(lm2, axis=1, keepdims=True)
    id2 = jnp.min(jnp.where(lm2 == m2, lane, big), axis=1, keepdims=True)
    w1p = 1.0 / (1.0 + jnp.exp(m2 - m1))
    w2p = 1.0 - w1p
    wtok_ref[...] = jnp.where(lane == id1, w1p,
                              jnp.where(lane == id2, w2p, 0.0))


def _ffn_body(wtok_ref, x_ref, w1_ref, w3_ref, w2_ref, out_ref, acc_ref):
    e = pl.program_id(1)
    x = x_ref[...].astype(jnp.bfloat16)
    h1 = jax.lax.dot_general(
        x, w1_ref[0], (((1,), (1,)), ((), ())),
        preferred_element_type=jnp.float32)
    h3 = jax.lax.dot_general(
        x, w3_ref[0], (((1,), (1,)), ((), ())),
        preferred_element_type=jnp.float32)
    inter = (h1 * jax.nn.sigmoid(h1)) * h3
    y = jax.lax.dot_general(
        inter.astype(jnp.bfloat16), w2_ref[0], (((1,), (1,)), ((), ())),
        preferred_element_type=jnp.float32)
    lane = jax.lax.broadcasted_iota(jnp.int32, (BM, EPAD), 1)
    wt = jnp.sum(jnp.where(lane == e, wtok_ref[...], 0.0), axis=1,
                 keepdims=True)
    contrib = y * wt

    @pl.when(e == 0)
    def _():
        acc_ref[...] = contrib

    @pl.when(e != 0)
    def _():
        acc_ref[...] = acc_ref[...] + contrib

    @pl.when(e == E - 1)
    def _():
        out_ref[...] = acc_ref[...]


@functools.partial(jax.jit, static_argnums=())
def kernel(hidden_states, gate_w, w1, w3, w2):
    b, s, h = hidden_states.shape
    hs = hidden_states.reshape(-1, h)
    gw_pad = jnp.zeros((EPAD, H), jnp.float32).at[:E].set(gate_w)

    logits_pad, wtok = pl.pallas_call(
        _router_body,
        out_shape=(
            jax.ShapeDtypeStruct((T, EPAD), jnp.float32),
            jax.ShapeDtypeStruct((T, EPAD), jnp.float32),
        ),
    )(hs, gw_pad)

    w1b = w1.astype(jnp.bfloat16)
    w3b = w3.astype(jnp.bfloat16)
    w2b = w2.astype(jnp.bfloat16)

    grid = (T // BM, E)
    final = pl.pallas_call(
        _ffn_body,
        grid=grid,
        in_specs=[
            pl.BlockSpec((BM, EPAD), lambda m, e: (m, 0)),
            pl.BlockSpec((BM, H), lambda m, e: (m, 0)),
            pl.BlockSpec((1, FFN, H), lambda m, e: (e, 0, 0)),
            pl.BlockSpec((1, FFN, H), lambda m, e: (e, 0, 0)),
            pl.BlockSpec((1, H, FFN), lambda m, e: (e, 0, 0)),
        ],
        out_specs=pl.BlockSpec((BM, H), lambda m, e: (m, 0)),
        out_shape=jax.ShapeDtypeStruct((T, H), jnp.float32),
        scratch_shapes=[pltpu.VMEM((BM, H), jnp.float32)],
        compiler_params=pltpu.CompilerParams(
            dimension_semantics=("arbitrary", "arbitrary")),
    )(wtok, hs, w1b, w3b, w2b)

    return (final.reshape(b, s, h), logits_pad[:, :E])
